# Initial kernel scaffold; baseline (speedup 1.0000x reference)
import jax
import jax.numpy as jnp
from jax.experimental import pallas as pl


def kernel(content, table, W1, b1, g1, be1, W2, b2, g2, be2):
    raise NotImplementedError("write your pallas kernel here")


# speedup vs baseline: 1.2629x; 1.2629x over previous
import functools

import jax
import jax.numpy as jnp
from jax import lax
from jax.experimental import pallas as pl
from jax.experimental.pallas import tpu as pltpu
from jax.experimental.pallas import tpu_sc as plsc

_V, _D, _H, _C = 1000000, 128, 64, 16
_B, _L = 4096, 200

_NC, _NS = 2, 16
_NW = _NC * _NS
_BPW = _B // _NW
_CH0, _CH1 = 128, 72
_NSL = _D // 16


def _make_sc_pool():
    mesh = plsc.VectorSubcoreMesh(core_axis_name="c", subcore_axis_name="s")

    @functools.partial(
        pl.kernel,
        mesh=mesh,
        out_type=jax.ShapeDtypeStruct((_B, _D), jnp.float32),
        scratch_types=[
            pltpu.VMEM((_BPW, _L), jnp.int32),
            pltpu.VMEM((_L, _D), jnp.float32),
            pltpu.VMEM((_BPW, _D), jnp.float32),
            pltpu.SemaphoreType.DMA,
        ],
    )
    def sc_pool(content_hbm, table_hbm, out_hbm, idx_v, rows_v, out_v, sem):
        wid = lax.axis_index("s") * _NC + lax.axis_index("c")
        base = wid * _BPW
        pltpu.sync_copy(content_hbm.at[pl.ds(base, _BPW)], idx_v)

        def body(row, carry):
            cp1 = pltpu.async_copy(
                table_hbm.at[idx_v.at[row, pl.ds(0, _CH0)]],
                rows_v.at[pl.ds(0, _CH0)], sem)
            cp2 = pltpu.async_copy(
                table_hbm.at[idx_v.at[row, pl.ds(_CH0, _CH1)]],
                rows_v.at[pl.ds(_CH0, _CH1)], sem)
            cp1.wait()
            cp2.wait()

            def acc_body(j, acc):
                return tuple(acc[c] + rows_v[j, pl.ds(c * 16, 16)]
                             for c in range(_NSL))

            zero = jnp.zeros((16,), jnp.float32)
            acc = lax.fori_loop(0, _L, acc_body, (zero,) * _NSL)
            for c in range(_NSL):
                out_v[row, pl.ds(c * 16, 16)] = acc[c]
            return carry

        lax.fori_loop(0, _BPW, body, 0)
        pltpu.sync_copy(out_v, out_hbm.at[pl.ds(base, _BPW)])

    return sc_pool


_sc_pool = _make_sc_pool()


def _mlp_body(x_ref, w1_ref, b1_ref, g1_ref, be1_ref, w2_ref, b2_ref, g2_ref,
              be2_ref, o_ref):
    x = x_ref[...] * (1.0 / _L)
    h = jnp.dot(x, w1_ref[...], preferred_element_type=jnp.float32) + b1_ref[...]
    m = jnp.mean(h, axis=0, keepdims=True)
    v = jnp.mean(jnp.square(h - m), axis=0, keepdims=True)
    h = g1_ref[...] * (h - m) * lax.rsqrt(v + 1e-5) + be1_ref[...]
    h = jnp.tanh(h)
    o = jnp.dot(h, w2_ref[...], preferred_element_type=jnp.float32) + b2_ref[...]
    m2 = jnp.mean(o, axis=0, keepdims=True)
    v2 = jnp.mean(jnp.square(o - m2), axis=0, keepdims=True)
    o = g2_ref[...] * (o - m2) * lax.rsqrt(v2 + 1e-5) + be2_ref[...]
    o_ref[...] = jax.nn.sigmoid(o)


_mlp = pl.pallas_call(
    _mlp_body,
    out_shape=jax.ShapeDtypeStruct((_B, _C), jnp.float32),
)


def kernel(content, table, W1, b1, g1, be1, W2, b2, g2, be2):
    xsum = _sc_pool(content, table)
    return _mlp(xsum, W1, b1.reshape(1, _H), g1.reshape(1, _H),
                be1.reshape(1, _H), W2, b2.reshape(1, _C), g2.reshape(1, _C),
                be2.reshape(1, _C))
